# baseline (speedup 1.0000x reference)
import functools

import jax
import jax.numpy as jnp
from jax import lax
from jax.experimental import pallas as pl
from jax.experimental.pallas import tpu as pltpu
from jax.experimental.pallas import tpu_sc as plsc

ROWS, COLS = 16384, 1024
N_CHUNKS = 8
CHUNK = ROWS // N_CHUNKS
SRC0 = 1 * CHUNK
SRC1 = (N_CHUNKS - 2) * CHUNK
OUT_ROWS = 2 * CHUNK

_NW = 32
ROWS_PER_W = OUT_ROWS // _NW

_mesh = plsc.VectorSubcoreMesh(core_axis_name="c", subcore_axis_name="s")

CHUNK_ROWS = 32
N_CHUNKS_PER_W = ROWS_PER_W // CHUNK_ROWS


@functools.partial(
    pl.kernel,
    mesh=_mesh,
    out_type=jax.ShapeDtypeStruct((OUT_ROWS, COLS), jnp.float32),
    scratch_types=[
        pltpu.VMEM((CHUNK_ROWS, COLS), jnp.float32),
        pltpu.VMEM((CHUNK_ROWS, COLS), jnp.float32),
        pltpu.SemaphoreType.DMA,
        pltpu.SemaphoreType.DMA,
        pltpu.SemaphoreType.DMA,
        pltpu.SemaphoreType.DMA,
    ],
)
def _zigzag_copy(flat_hbm, out_hbm, buf_a, buf_b, sem_ga, sem_gb, sem_sa, sem_sb):
    wid = lax.axis_index("s") * 2 + lax.axis_index("c")
    dst0 = wid * ROWS_PER_W
    src0 = jnp.where(dst0 < CHUNK, SRC0 + dst0, SRC1 + (dst0 - CHUNK))

    bufs = (buf_a, buf_b)
    gsems = (sem_ga, sem_gb)
    ssems = (sem_sa, sem_sb)

    def gather(i):
        return pltpu.async_copy(
            flat_hbm.at[pl.ds(src0 + i * CHUNK_ROWS, CHUNK_ROWS), :],
            bufs[i % 2],
            gsems[i % 2],
        )

    def scatter(i):
        return pltpu.async_copy(
            bufs[i % 2],
            out_hbm.at[pl.ds(dst0 + i * CHUNK_ROWS, CHUNK_ROWS), :],
            ssems[i % 2],
        )

    gathers = [None] * N_CHUNKS_PER_W
    scatters = [None] * N_CHUNKS_PER_W
    gathers[0] = gather(0)
    gathers[1] = gather(1)
    for i in range(N_CHUNKS_PER_W):
        gathers[i].wait()
        scatters[i] = scatter(i)
        nxt = i + 2
        if nxt < N_CHUNKS_PER_W:
            scatters[nxt - 2].wait()
            gathers[nxt] = gather(nxt)
    scatters[N_CHUNKS_PER_W - 2].wait()
    scatters[N_CHUNKS_PER_W - 1].wait()


TC_CH = 256
TC_NBUF = 16
TC_NCHUNKS = OUT_ROWS // TC_CH


def _tc_src(i):
    dst = i * TC_CH
    return SRC0 + dst if dst < CHUNK else SRC1 + (dst - CHUNK)


def _tc_copy_body(in_ref, out_ref, *scratch):
    bufs = scratch[:TC_NBUF]
    gsems = scratch[TC_NBUF:2 * TC_NBUF]
    ssems = scratch[2 * TC_NBUF:]

    def gather(i):
        return pltpu.make_async_copy(
            in_ref.at[pl.ds(_tc_src(i), TC_CH), :], bufs[i % TC_NBUF],
            gsems[i % TC_NBUF])

    def scatter(i):
        return pltpu.make_async_copy(
            bufs[i % TC_NBUF], out_ref.at[pl.ds(i * TC_CH, TC_CH), :],
            ssems[i % TC_NBUF])

    scatters = [None] * TC_NCHUNKS
    gathers = [None] * TC_NCHUNKS
    for i in range(min(TC_NBUF, TC_NCHUNKS)):
        gathers[i] = gather(i)
        gathers[i].start()
    for i in range(TC_NCHUNKS):
        gathers[i].wait()
        scatters[i] = scatter(i)
        scatters[i].start()
        nxt = i + TC_NBUF
        if nxt < TC_NCHUNKS:
            scatters[i].wait()
            gathers[nxt] = gather(nxt)
            gathers[nxt].start()
    for i in range(max(0, TC_NCHUNKS - TC_NBUF), TC_NCHUNKS):
        scatters[i].wait()


def _tc_copy(flat):
    scratch = (
        [pltpu.VMEM((TC_CH, COLS), jnp.float32)] * TC_NBUF
        + [pltpu.SemaphoreType.DMA] * (2 * TC_NBUF)
    )
    return pl.pallas_call(
        _tc_copy_body,
        in_specs=[pl.BlockSpec(memory_space=pl.ANY)],
        out_specs=pl.BlockSpec(memory_space=pl.ANY),
        out_shape=jax.ShapeDtypeStruct((OUT_ROWS, COLS), jnp.float32),
        scratch_shapes=scratch,
    )(flat)


def kernel(flat):
    return _tc_copy(flat)


# speedup vs baseline: 1.0200x; 1.0200x over previous
import functools

import jax
import jax.numpy as jnp
from jax import lax
from jax.experimental import pallas as pl
from jax.experimental.pallas import tpu as pltpu
from jax.experimental.pallas import tpu_sc as plsc

ROWS, COLS = 16384, 1024
N_CHUNKS = 8
CHUNK = ROWS // N_CHUNKS
SRC0 = 1 * CHUNK
SRC1 = (N_CHUNKS - 2) * CHUNK
OUT_ROWS = 2 * CHUNK

_NW = 32
ROWS_PER_W = OUT_ROWS // _NW

_mesh = plsc.VectorSubcoreMesh(core_axis_name="c", subcore_axis_name="s")

CHUNK_ROWS = 32
N_CHUNKS_PER_W = ROWS_PER_W // CHUNK_ROWS


@functools.partial(
    pl.kernel,
    mesh=_mesh,
    out_type=jax.ShapeDtypeStruct((OUT_ROWS, COLS), jnp.float32),
    scratch_types=[
        pltpu.VMEM((CHUNK_ROWS, COLS), jnp.float32),
        pltpu.VMEM((CHUNK_ROWS, COLS), jnp.float32),
        pltpu.SemaphoreType.DMA,
        pltpu.SemaphoreType.DMA,
        pltpu.SemaphoreType.DMA,
        pltpu.SemaphoreType.DMA,
    ],
)
def _zigzag_copy(flat_hbm, out_hbm, buf_a, buf_b, sem_ga, sem_gb, sem_sa, sem_sb):
    wid = lax.axis_index("s") * 2 + lax.axis_index("c")
    dst0 = wid * ROWS_PER_W
    src0 = jnp.where(dst0 < CHUNK, SRC0 + dst0, SRC1 + (dst0 - CHUNK))

    bufs = (buf_a, buf_b)
    gsems = (sem_ga, sem_gb)
    ssems = (sem_sa, sem_sb)

    def gather(i):
        return pltpu.async_copy(
            flat_hbm.at[pl.ds(src0 + i * CHUNK_ROWS, CHUNK_ROWS), :],
            bufs[i % 2],
            gsems[i % 2],
        )

    def scatter(i):
        return pltpu.async_copy(
            bufs[i % 2],
            out_hbm.at[pl.ds(dst0 + i * CHUNK_ROWS, CHUNK_ROWS), :],
            ssems[i % 2],
        )

    gathers = [None] * N_CHUNKS_PER_W
    scatters = [None] * N_CHUNKS_PER_W
    gathers[0] = gather(0)
    gathers[1] = gather(1)
    for i in range(N_CHUNKS_PER_W):
        gathers[i].wait()
        scatters[i] = scatter(i)
        nxt = i + 2
        if nxt < N_CHUNKS_PER_W:
            scatters[nxt - 2].wait()
            gathers[nxt] = gather(nxt)
    scatters[N_CHUNKS_PER_W - 2].wait()
    scatters[N_CHUNKS_PER_W - 1].wait()


TC_CH = 1024
TC_NBUF = 4
TC_NCHUNKS = OUT_ROWS // TC_CH


def _tc_src(i):
    dst = i * TC_CH
    return SRC0 + dst if dst < CHUNK else SRC1 + (dst - CHUNK)


def _tc_copy_body(in_ref, out_ref, *scratch):
    bufs = scratch[:TC_NBUF]
    gsems = scratch[TC_NBUF:2 * TC_NBUF]
    ssems = scratch[2 * TC_NBUF:]

    def gather(i):
        return pltpu.make_async_copy(
            in_ref.at[pl.ds(_tc_src(i), TC_CH), :], bufs[i % TC_NBUF],
            gsems[i % TC_NBUF])

    def scatter(i):
        return pltpu.make_async_copy(
            bufs[i % TC_NBUF], out_ref.at[pl.ds(i * TC_CH, TC_CH), :],
            ssems[i % TC_NBUF])

    scatters = [None] * TC_NCHUNKS
    gathers = [None] * TC_NCHUNKS
    for i in range(min(TC_NBUF, TC_NCHUNKS)):
        gathers[i] = gather(i)
        gathers[i].start()
    for i in range(TC_NCHUNKS):
        gathers[i].wait()
        scatters[i] = scatter(i)
        scatters[i].start()
        nxt = i + TC_NBUF
        if nxt < TC_NCHUNKS:
            scatters[i].wait()
            gathers[nxt] = gather(nxt)
            gathers[nxt].start()
    for i in range(max(0, TC_NCHUNKS - TC_NBUF), TC_NCHUNKS):
        scatters[i].wait()


def _tc_copy(flat):
    scratch = (
        [pltpu.VMEM((TC_CH, COLS), jnp.float32)] * TC_NBUF
        + [pltpu.SemaphoreType.DMA] * (2 * TC_NBUF)
    )
    return pl.pallas_call(
        _tc_copy_body,
        in_specs=[pl.BlockSpec(memory_space=pl.ANY)],
        out_specs=pl.BlockSpec(memory_space=pl.ANY),
        out_shape=jax.ShapeDtypeStruct((OUT_ROWS, COLS), jnp.float32),
        scratch_shapes=scratch,
    )(flat)


def kernel(flat):
    return _tc_copy(flat)


# speedup vs baseline: 1.0755x; 1.0544x over previous
import functools

import jax
import jax.numpy as jnp
from jax import lax
from jax.experimental import pallas as pl
from jax.experimental.pallas import tpu as pltpu
from jax.experimental.pallas import tpu_sc as plsc

ROWS, COLS = 16384, 1024
N_CHUNKS = 8
CHUNK = ROWS // N_CHUNKS
SRC0 = 1 * CHUNK
SRC1 = (N_CHUNKS - 2) * CHUNK
OUT_ROWS = 2 * CHUNK

_NW = 32
ROWS_PER_W = OUT_ROWS // _NW

_mesh = plsc.VectorSubcoreMesh(core_axis_name="c", subcore_axis_name="s")

CHUNK_ROWS = 32
N_CHUNKS_PER_W = ROWS_PER_W // CHUNK_ROWS


@functools.partial(
    pl.kernel,
    mesh=_mesh,
    out_type=jax.ShapeDtypeStruct((OUT_ROWS, COLS), jnp.float32),
    scratch_types=[
        pltpu.VMEM((CHUNK_ROWS, COLS), jnp.float32),
        pltpu.VMEM((CHUNK_ROWS, COLS), jnp.float32),
        pltpu.SemaphoreType.DMA,
        pltpu.SemaphoreType.DMA,
        pltpu.SemaphoreType.DMA,
        pltpu.SemaphoreType.DMA,
    ],
)
def _zigzag_copy(flat_hbm, out_hbm, buf_a, buf_b, sem_ga, sem_gb, sem_sa, sem_sb):
    wid = lax.axis_index("s") * 2 + lax.axis_index("c")
    dst0 = wid * ROWS_PER_W
    src0 = jnp.where(dst0 < CHUNK, SRC0 + dst0, SRC1 + (dst0 - CHUNK))

    bufs = (buf_a, buf_b)
    gsems = (sem_ga, sem_gb)
    ssems = (sem_sa, sem_sb)

    def gather(i):
        return pltpu.async_copy(
            flat_hbm.at[pl.ds(src0 + i * CHUNK_ROWS, CHUNK_ROWS), :],
            bufs[i % 2],
            gsems[i % 2],
        )

    def scatter(i):
        return pltpu.async_copy(
            bufs[i % 2],
            out_hbm.at[pl.ds(dst0 + i * CHUNK_ROWS, CHUNK_ROWS), :],
            ssems[i % 2],
        )

    gathers = [None] * N_CHUNKS_PER_W
    scatters = [None] * N_CHUNKS_PER_W
    gathers[0] = gather(0)
    gathers[1] = gather(1)
    for i in range(N_CHUNKS_PER_W):
        gathers[i].wait()
        scatters[i] = scatter(i)
        nxt = i + 2
        if nxt < N_CHUNKS_PER_W:
            scatters[nxt - 2].wait()
            gathers[nxt] = gather(nxt)
    scatters[N_CHUNKS_PER_W - 2].wait()
    scatters[N_CHUNKS_PER_W - 1].wait()


TC_CHUNK_ROWS = [128, 128, 256, 512, 1024, 1024, 512, 256, 128, 128]
assert sum(TC_CHUNK_ROWS) == OUT_ROWS


def _tc_chunks():
    offs, o = [], 0
    for n in TC_CHUNK_ROWS:
        src = SRC0 + o if o < CHUNK else SRC1 + (o - CHUNK)
        offs.append((src, o, n))
        o += n
    return offs


def _tc_copy_body(in_ref, out_ref, *scratch):
    n = len(TC_CHUNK_ROWS)
    bufs = scratch[:n]
    gsems = scratch[n:2 * n]
    ssems = scratch[2 * n:]
    chunks = _tc_chunks()

    gathers = []
    for i, (src, dst, rows) in enumerate(chunks):
        c = pltpu.make_async_copy(
            in_ref.at[pl.ds(src, rows), :], bufs[i], gsems[i])
        c.start()
        gathers.append(c)
    scatters = []
    for i, (src, dst, rows) in enumerate(chunks):
        gathers[i].wait()
        c = pltpu.make_async_copy(
            bufs[i], out_ref.at[pl.ds(dst, rows), :], ssems[i])
        c.start()
        scatters.append(c)
    for c in scatters:
        c.wait()


def _tc_copy(flat):
    n = len(TC_CHUNK_ROWS)
    scratch = (
        [pltpu.VMEM((rows, COLS), jnp.float32) for rows in TC_CHUNK_ROWS]
        + [pltpu.SemaphoreType.DMA] * (2 * n)
    )
    return pl.pallas_call(
        _tc_copy_body,
        in_specs=[pl.BlockSpec(memory_space=pl.ANY)],
        out_specs=pl.BlockSpec(memory_space=pl.ANY),
        out_shape=jax.ShapeDtypeStruct((OUT_ROWS, COLS), jnp.float32),
        scratch_shapes=scratch,
    )(flat)


def kernel(flat):
    return _tc_copy(flat)
